# baseline (speedup 1.0000x reference)
import jax
import jax.numpy as jnp
from jax import lax
from jax.experimental import pallas as pl
from jax.experimental.pallas import tpu as pltpu
from jax.experimental.pallas import tpu_sc as plsc

_NC = 2
_NS = 16
_NW = _NC * _NS

_B, _N, _S, _D = 4, 16, 2048, 128
_ROW = _S * _D
_PIECE = 65536
_ROWP = _ROW // _PIECE
_NJ = 60
_NQ = _B * _NJ * _ROWP
_QPW = _NQ // _NW


def _src_offset(q):
    r, p = q // _ROWP, q % _ROWP
    b, j = r // _NJ, r % _NJ
    h, m = j // 30, j % 30
    head = jnp.where(h == 0, m, 3 * (m % 3) + m // 3)
    t, g = (m - 9) % 3, (m - 9) // 3
    pair = jnp.where(h == 0, 6 + t, 2 + 3 * t)
    tail = jnp.where(t < 2, pair, g + 9)
    idx = jnp.where(m < 9, head, tail)
    return ((b * _N + idx) * _ROWP + p) * _PIECE


def _body(x_hbm, out_hbm, buf0, buf1, sg0, sg1, ss0, ss1):
    c = lax.axis_index("c")
    s = lax.axis_index("s")
    w = s * _NC + c
    bufs = (buf0.at[s], buf1)
    sgs, sss = (sg0, sg1), (ss0, ss1)

    def start_gather(k, b):
        off = pl.multiple_of(_src_offset(w * _QPW + k), _PIECE)
        pltpu.async_copy(x_hbm.at[pl.ds(off, _PIECE)], bufs[b], sgs[b])

    def wait_gather(b):
        pltpu.make_async_copy(x_hbm.at[pl.ds(0, _PIECE)], bufs[b],
                              sgs[b]).wait()

    def start_store(k, b):
        off = pl.multiple_of((w * _QPW + k) * _PIECE, _PIECE)
        pltpu.async_copy(bufs[b], out_hbm.at[pl.ds(off, _PIECE)], sss[b])

    def wait_store(b):
        pltpu.make_async_copy(bufs[b], out_hbm.at[pl.ds(0, _PIECE)],
                              sss[b]).wait()

    start_gather(0, 0)
    wait_gather(0)
    start_gather(1, 1)
    start_store(0, 0)

    @pl.loop(1, _QPW // 2)
    def _(t):
        k1 = 2 * t - 1
        wait_gather(1)
        wait_store(0)
        start_gather(k1 + 1, 0)
        start_store(k1, 1)
        k2 = 2 * t
        wait_gather(0)
        wait_store(1)
        start_gather(k2 + 1, 1)
        start_store(k2, 0)

    wait_gather(1)
    wait_store(0)
    start_store(_QPW - 1, 1)
    wait_store(1)


@jax.jit
def kernel(x):
    b, n, s, d = x.shape
    x1 = x.reshape(-1)
    mesh = plsc.VectorSubcoreMesh(core_axis_name="c", subcore_axis_name="s")
    out = pl.kernel(
        _body,
        out_type=jax.ShapeDtypeStruct((_NQ * _PIECE,), jnp.float32),
        mesh=mesh,
        scratch_types=[
            pltpu.VMEM_SHARED((_NS, _PIECE), jnp.float32),
            pltpu.VMEM((_PIECE,), jnp.float32),
            pltpu.SemaphoreType.DMA,
            pltpu.SemaphoreType.DMA,
            pltpu.SemaphoreType.DMA,
            pltpu.SemaphoreType.DMA,
        ],
    )(x1)
    return out.reshape(b, _NJ // 3, 3, s, d)


# speedup vs baseline: 1.0254x; 1.0254x over previous
import jax
import jax.numpy as jnp
from jax import lax
from jax.experimental import pallas as pl
from jax.experimental.pallas import tpu as pltpu
from jax.experimental.pallas import tpu_sc as plsc

_NC = 2
_NS = 16
_NW = _NC * _NS

_B, _N, _S, _D = 4, 16, 2048, 128
_ROW = _S * _D
_PIECE = 32768
_ROWP = _ROW // _PIECE
_NJ = 60
_NQ = _B * _NJ * _ROWP
_QPW = _NQ // _NW


def _src_offset(q):
    r, p = q // _ROWP, q % _ROWP
    b, j = r // _NJ, r % _NJ
    h, m = j // 30, j % 30
    head = jnp.where(h == 0, m, 3 * (m % 3) + m // 3)
    t, g = (m - 9) % 3, (m - 9) // 3
    pair = jnp.where(h == 0, 6 + t, 2 + 3 * t)
    tail = jnp.where(t < 2, pair, g + 9)
    idx = jnp.where(m < 9, head, tail)
    return ((b * _N + idx) * _ROWP + p) * _PIECE


def _body(x_hbm, out_hbm, buf0, buf1, sg0, sg1, ss0, ss1):
    c = lax.axis_index("c")
    s = lax.axis_index("s")
    w = s * _NC + c
    bufs = (buf0.at[s], buf1.at[s])
    sgs, sss = (sg0, sg1), (ss0, ss1)

    def start_gather(k, b):
        off = pl.multiple_of(_src_offset(w * _QPW + k), _PIECE)
        pltpu.async_copy(x_hbm.at[pl.ds(off, _PIECE)], bufs[b], sgs[b])

    def wait_gather(b):
        pltpu.make_async_copy(x_hbm.at[pl.ds(0, _PIECE)], bufs[b],
                              sgs[b]).wait()

    def start_store(k, b):
        off = pl.multiple_of((w * _QPW + k) * _PIECE, _PIECE)
        pltpu.async_copy(bufs[b], out_hbm.at[pl.ds(off, _PIECE)], sss[b])

    def wait_store(b):
        pltpu.make_async_copy(bufs[b], out_hbm.at[pl.ds(0, _PIECE)],
                              sss[b]).wait()

    start_gather(0, 0)
    wait_gather(0)
    start_gather(1, 1)
    start_store(0, 0)

    @pl.loop(1, _QPW // 2)
    def _(t):
        k1 = 2 * t - 1
        wait_gather(1)
        wait_store(0)
        start_gather(k1 + 1, 0)
        start_store(k1, 1)
        k2 = 2 * t
        wait_gather(0)
        wait_store(1)
        start_gather(k2 + 1, 1)
        start_store(k2, 0)

    wait_gather(1)
    wait_store(0)
    start_store(_QPW - 1, 1)
    wait_store(1)


@jax.jit
def kernel(x):
    b, n, s, d = x.shape
    x1 = x.reshape(-1)
    mesh = plsc.VectorSubcoreMesh(core_axis_name="c", subcore_axis_name="s")
    out = pl.kernel(
        _body,
        out_type=jax.ShapeDtypeStruct((_NQ * _PIECE,), jnp.float32),
        mesh=mesh,
        scratch_types=[
            pltpu.VMEM_SHARED((_NS, _PIECE), jnp.float32),
            pltpu.VMEM_SHARED((_NS, _PIECE), jnp.float32),
            pltpu.SemaphoreType.DMA,
            pltpu.SemaphoreType.DMA,
            pltpu.SemaphoreType.DMA,
            pltpu.SemaphoreType.DMA,
        ],
    )(x1)
    return out.reshape(b, _NJ // 3, 3, s, d)


# speedup vs baseline: 1.3545x; 1.3209x over previous
import jax
import jax.numpy as jnp
from jax import lax
from jax.experimental import pallas as pl
from jax.experimental.pallas import tpu as pltpu
from jax.experimental.pallas import tpu_sc as plsc

_NC = 2
_NS = 16
_NW = _NC * _NS

_B, _N, _S, _D = 4, 16, 2048, 128
_ROW = _S * _D
_PIECE = 65536
_ROWP = _ROW // _PIECE
_NJ = 60
_NQ = _B * _NJ * _ROWP

_NSTEP = 2 * _ROWP + 2


def _step_offsets(w, k):
    if k < 2 * _ROWP:
        u, p = k // _ROWP, k % _ROWP
        n = 2 * w + u
        b, i = n // _N, n % _N
        src = ((b * _N + i) * _ROWP + p) * _PIECE
        j1 = jnp.where(i < 9, i, 3 * i - 16)
        j2 = jnp.where(i < 9, 30 + 3 * (i % 3) + i // 3, 3 * i + 14)
        dsts = [((b * _NJ + j) * _ROWP + p) * _PIECE for j in (j1, j2)]
    else:
        h, half = w // 2, w % 2
        p = 2 * half + (k - 2 * _ROWP)
        b, e = h // 4, h % 4
        src_row = jnp.where(e < 2, 6 + e, 3 * e - 4)
        j0 = jnp.where(e < 2, 9 + e, 37 + e)
        src = ((b * _N + src_row) * _ROWP + p) * _PIECE
        dsts = [((b * _NJ + j0 + 3 * t) * _ROWP + p) * _PIECE
                for t in range(7)]
    return src, dsts


def _body(x_hbm, out_hbm, buf0, buf1, sg0, sg1, ss0, ss1):
    c = lax.axis_index("c")
    s = lax.axis_index("s")
    w = s * _NC + c
    bufs = (buf0.at[s], buf1)
    sgs, sss = (sg0, sg1), (ss0, ss1)

    def n_stores(k):
        return 2 if k < 2 * _ROWP else 7

    def start_gather(k):
        src, _ = _step_offsets(w, k)
        b = k % 2
        pltpu.async_copy(x_hbm.at[pl.ds(pl.multiple_of(src, _PIECE), _PIECE)],
                         bufs[b], sgs[b])

    def wait_gather(k):
        b = k % 2
        pltpu.make_async_copy(x_hbm.at[pl.ds(0, _PIECE)], bufs[b],
                              sgs[b]).wait()

    def start_stores(k):
        _, dsts = _step_offsets(w, k)
        b = k % 2
        for d in dsts:
            pltpu.async_copy(
                bufs[b], out_hbm.at[pl.ds(pl.multiple_of(d, _PIECE), _PIECE)],
                sss[b])

    def wait_stores(k):
        b = k % 2
        for _ in range(n_stores(k)):
            pltpu.make_async_copy(bufs[b], out_hbm.at[pl.ds(0, _PIECE)],
                                  sss[b]).wait()

    start_gather(0)
    for k in range(_NSTEP):
        wait_gather(k)
        start_stores(k)
        if k + 1 < _NSTEP:
            if k >= 1:
                wait_stores(k - 1)
            start_gather(k + 1)
    wait_stores(_NSTEP - 2)
    wait_stores(_NSTEP - 1)


@jax.jit
def kernel(x):
    b, n, s, d = x.shape
    x1 = x.reshape(-1)
    mesh = plsc.VectorSubcoreMesh(core_axis_name="c", subcore_axis_name="s")
    out = pl.kernel(
        _body,
        out_type=jax.ShapeDtypeStruct((_NQ * _PIECE,), jnp.float32),
        mesh=mesh,
        scratch_types=[
            pltpu.VMEM_SHARED((_NS, _PIECE), jnp.float32),
            pltpu.VMEM((_PIECE,), jnp.float32),
            pltpu.SemaphoreType.DMA,
            pltpu.SemaphoreType.DMA,
            pltpu.SemaphoreType.DMA,
            pltpu.SemaphoreType.DMA,
        ],
    )(x1)
    return out.reshape(b, _NJ // 3, 3, s, d)


# speedup vs baseline: 1.3686x; 1.0104x over previous
import jax
import jax.numpy as jnp
from jax import lax
from jax.experimental import pallas as pl
from jax.experimental.pallas import tpu as pltpu
from jax.experimental.pallas import tpu_sc as plsc

_NC = 2
_NS = 16
_NW = _NC * _NS

_B, _N, _S, _D = 4, 16, 2048, 128
_ROW = _S * _D
_PIECE = 65536
_ROWP = _ROW // _PIECE
_NJ = 60
_NQ = _B * _NJ * _ROWP

_NSTEP = 2 * _ROWP + 2


def _step_offsets(w, k):
    if k < 2 * _ROWP:
        u, p = k // _ROWP, k % _ROWP
        n = 2 * w + u
        b, i = n // _N, n % _N
        src = ((b * _N + i) * _ROWP + p) * _PIECE
        j1 = jnp.where(i < 9, i, 3 * i - 16)
        j2 = jnp.where(i < 9, 30 + 3 * (i % 3) + i // 3, 3 * i + 14)
        dsts = [((b * _NJ + j) * _ROWP + p) * _PIECE for j in (j1, j2)]
    else:
        h, half = w // 2, w % 2
        p = 2 * half + (k - 2 * _ROWP)
        b, e = h // 4, h % 4
        src_row = jnp.where(e < 2, 6 + e, 3 * e - 4)
        j0 = jnp.where(e < 2, 9 + e, 37 + e)
        src = ((b * _N + src_row) * _ROWP + p) * _PIECE
        dsts = [((b * _NJ + j0 + 3 * t) * _ROWP + p) * _PIECE
                for t in range(7)]
    return src, dsts


def _body(x_hbm, out_hbm, buf0, buf1, sg0, sg1, ss0, ss1):
    c = lax.axis_index("c")
    s = lax.axis_index("s")
    w = s * _NC + c
    bufs = (buf0.at[s], buf1)
    sgs, sss = (sg0, sg1), (ss0, ss1)

    def n_stores(k):
        return 2 if k < 2 * _ROWP else 7

    def start_gather(k):
        src, _ = _step_offsets(w, k)
        b = k % 2
        pltpu.async_copy(x_hbm.at[pl.ds(pl.multiple_of(src, _PIECE), _PIECE)],
                         bufs[b], sgs[b])

    def wait_gather(k):
        b = k % 2
        pltpu.make_async_copy(x_hbm.at[pl.ds(0, _PIECE)], bufs[b],
                              sgs[b]).wait()

    def start_stores(k):
        _, dsts = _step_offsets(w, k)
        b = k % 2
        for d in dsts:
            pltpu.async_copy(
                bufs[b], out_hbm.at[pl.ds(pl.multiple_of(d, _PIECE), _PIECE)],
                sss[b])

    def wait_stores(k):
        b = k % 2
        for _ in range(n_stores(k)):
            pltpu.make_async_copy(bufs[b], out_hbm.at[pl.ds(0, _PIECE)],
                                  sss[b]).wait()

    order = [2 * _ROWP, 2 * _ROWP + 1] + list(range(2 * _ROWP))
    start_gather(order[0])
    for p, k in enumerate(order):
        wait_gather(k)
        start_stores(k)
        if p + 1 < _NSTEP:
            if p >= 1:
                wait_stores(order[p - 1])
            start_gather(order[p + 1])
    wait_stores(order[_NSTEP - 2])
    wait_stores(order[_NSTEP - 1])


@jax.jit
def kernel(x):
    b, n, s, d = x.shape
    x1 = x.reshape(-1)
    mesh = plsc.VectorSubcoreMesh(core_axis_name="c", subcore_axis_name="s")
    out = pl.kernel(
        _body,
        out_type=jax.ShapeDtypeStruct((_NQ * _PIECE,), jnp.float32),
        mesh=mesh,
        scratch_types=[
            pltpu.VMEM_SHARED((_NS, _PIECE), jnp.float32),
            pltpu.VMEM((_PIECE,), jnp.float32),
            pltpu.SemaphoreType.DMA,
            pltpu.SemaphoreType.DMA,
            pltpu.SemaphoreType.DMA,
            pltpu.SemaphoreType.DMA,
        ],
    )(x1)
    return out.reshape(b, _NJ // 3, 3, s, d)
